# baseline (speedup 1.0000x reference)
import functools

import jax
import jax.numpy as jnp
from jax import lax
from jax.experimental import pallas as pl
from jax.experimental.pallas import tpu as pltpu
from jax.experimental.pallas import tpu_sc as plsc

B = 16384
D = 64


_NC, _NS = 2, 16
_NW = _NC * _NS
_BPW = B // _NW
_IDXW = 128
_NG = _BPW // _IDXW


def _gather_body(uid_hbm, iid_hbm, utab_hbm, itab_hbm, uout_hbm, iout_hbm,
                 uidx_v, iidx_v, rows_v, sem):
    wid = lax.axis_index("s") * _NC + lax.axis_index("c")
    base = wid * _BPW
    rbase = wid * _NG
    pltpu.sync_copy(uid_hbm.at[pl.ds(rbase, _NG)], uidx_v)
    pltpu.sync_copy(iid_hbm.at[pl.ds(rbase, _NG)], iidx_v)
    for idx_v, tab_hbm, out_hbm in ((uidx_v, utab_hbm, uout_hbm),
                                    (iidx_v, itab_hbm, iout_hbm)):
        handles = [pltpu.make_async_copy(
            tab_hbm.at[idx_v.at[j]],
            rows_v.at[pl.ds(j * _IDXW, _IDXW)], sem) for j in range(_NG)]
        for h in handles:
            h.start()
        for h in handles:
            h.wait()
        pltpu.sync_copy(rows_v, out_hbm.at[pl.ds(base, _BPW)])


@functools.cache
def _make_gather():
    return pl.kernel(
        _gather_body,
        out_type=(
            jax.ShapeDtypeStruct((B, 128), jnp.float32),
            jax.ShapeDtypeStruct((B, 128), jnp.float32),
        ),
        mesh=plsc.VectorSubcoreMesh(core_axis_name="c", subcore_axis_name="s",
                                    num_cores=_NC, num_subcores=_NS),
        scratch_types=[
            pltpu.VMEM((_NG, _IDXW), jnp.int32),
            pltpu.VMEM((_NG, _IDXW), jnp.int32),
            pltpu.VMEM((_BPW, 128), jnp.float32),
            pltpu.SemaphoreType.DMA,
        ],
    )


_BSZ = 1024


def _towers_body(ue_ref, ie_ref, up_ref, ip_ref,
                 uW1, ub1, uW2, ub2, uPW, uPb,
                 iW1, ib1, iW2, ib2, iPW, iPb, rW, rb,
                 uo_ref, io_ref, r_ref):
    uodd = (up_ref[...] & 1) == 1
    iodd = (ip_ref[...] & 1) == 1
    ue = jnp.where(uodd, ue_ref[:, D:], ue_ref[:, :D])
    ie = jnp.where(iodd, ie_ref[:, D:], ie_ref[:, :D])
    f32 = jnp.float32
    u = jnp.maximum(jnp.dot(ue, uW1[...], preferred_element_type=f32) + ub1[...], 0.0)
    u = jnp.maximum(jnp.dot(u, uW2[...], preferred_element_type=f32) + ub2[...], 0.0)
    uo = jnp.dot(u, uPW[...], preferred_element_type=f32) + uPb[...]
    it = jnp.maximum(jnp.dot(ie, iW1[...], preferred_element_type=f32) + ib1[...], 0.0)
    it = jnp.maximum(jnp.dot(it, iW2[...], preferred_element_type=f32) + ib2[...], 0.0)
    io = jnp.dot(it, iPW[...], preferred_element_type=f32) + iPb[...]
    uo_ref[...] = uo
    io_ref[...] = io
    inter = uo * io
    r = jnp.sum(inter * rW[...], axis=1, keepdims=True) + rb[...]
    r_ref[...] = jax.nn.sigmoid(r) * 5.0


def _towers(ue, ie, up, ip, uW1, ub1, uW2, ub2, uPW, uPb,
            iW1, ib1, iW2, ib2, iPW, iPb, rW, rb):
    full = lambda s: pl.BlockSpec(s, lambda i: (0, 0))
    bspec = pl.BlockSpec((_BSZ, 128), lambda i: (i, 0))
    pspec = pl.BlockSpec((_BSZ, 1), lambda i: (i, 0))
    ospec = pl.BlockSpec((_BSZ, D), lambda i: (i, 0))
    H1, H2 = uW1.shape[1], uW2.shape[1]
    return pl.pallas_call(
        _towers_body,
        grid=(B // _BSZ,),
        in_specs=[
            bspec, bspec, pspec, pspec,
            full((D, H1)), full((1, H1)), full((H1, H2)), full((1, H2)),
            full((H2, D)), full((1, D)),
            full((D, H1)), full((1, H1)), full((H1, H2)), full((1, H2)),
            full((H2, D)), full((1, D)),
            full((1, D)), full((1, 1)),
        ],
        out_specs=[
            ospec, ospec,
            pl.BlockSpec((_BSZ, 1), lambda i: (i, 0)),
        ],
        out_shape=[
            jax.ShapeDtypeStruct((B, D), jnp.float32),
            jax.ShapeDtypeStruct((B, D), jnp.float32),
            jax.ShapeDtypeStruct((B, 1), jnp.float32),
        ],
    )(ue, ie, up, ip, uW1, ub1, uW2, ub2, uPW, uPb,
      iW1, ib1, iW2, ib2, iPW, iPb, rW, rb)


def kernel(user_id, item_id, user_table, item_table, uW1, ub1, uW2, ub2, uPW, uPb,
           iW1, ib1, iW2, ib2, iPW, iPb, rW, rb):
    uid2 = (user_id >> 1).reshape(B // _IDXW, _IDXW)
    iid2 = (item_id >> 1).reshape(B // _IDXW, _IDXW)
    ut2 = user_table.reshape(-1, 128)
    it2 = item_table.reshape(-1, 128)
    ue, ie = _make_gather()(uid2, iid2, ut2, it2)
    return _towers(ue, ie, user_id.reshape(B, 1), item_id.reshape(B, 1),
                   uW1, ub1.reshape(1, -1), uW2, ub2.reshape(1, -1),
                   uPW, uPb.reshape(1, -1),
                   iW1, ib1.reshape(1, -1), iW2, ib2.reshape(1, -1),
                   iPW, iPb.reshape(1, -1),
                   rW.reshape(1, -1), rb.reshape(1, 1))
